# Initial kernel scaffold; baseline (speedup 1.0000x reference)
import jax
import jax.numpy as jnp
from jax.experimental import pallas as pl


def kernel(preds, anchors, image_size):
    raise NotImplementedError("write your pallas kernel here")


# speedup vs baseline: 3.3510x; 3.3510x over previous
import functools

import jax
import jax.numpy as jnp
from jax.experimental import pallas as pl
from jax.experimental.pallas import tpu as pltpu

_CONF = 0.2
_IOU = 0.6
_MAXDET = 300
_MAXWH = 4096.0
_SELW = 512


def _decode_body(params_ref, preds_ref, x1_ref, y1_ref, x2_ref, y2_ref,
                 cf_ref, cl_ref, *, nc, H, W):
    l = pl.program_id(0)
    a = pl.program_id(2)
    na = pl.num_programs(2)
    h = l * na + a
    aw = params_ref[h, 0]
    ah = params_ref[h, 1]
    sw = params_ref[h, 2]
    sh = params_ref[h, 3]

    p = preds_ref[0, 0, 0]
    s = jax.nn.sigmoid(p)

    ni = jax.lax.broadcasted_iota(jnp.int32, (1, H * W), 1)
    xg = (ni % W).astype(jnp.float32)
    yg = (ni // W).astype(jnp.float32)

    xc = (s[0:1, :] * 3.0 - 1.0 + xg) * sw
    yc = (s[1:2, :] * 3.0 - 1.0 + yg) * sh
    w = ((s[2:3, :] * 2.0) ** 2 * aw) * sw
    hh = ((s[3:4, :] * 2.0) ** 2 * ah) * sh

    x1_ref[0, 0] = xc - w / 2.0
    y1_ref[0, 0] = yc - hh / 2.0
    x2_ref[0, 0] = xc + w / 2.0
    y2_ref[0, 0] = yc + hh / 2.0

    obj = s[4:5, :]
    cls_s = s[5:, :] * obj
    conf = jnp.max(cls_s, axis=0, keepdims=True)
    kio = jax.lax.broadcasted_iota(jnp.int32, (nc - 5, H * W), 0)
    cls_i = jnp.min(jnp.where(cls_s == conf, kio, jnp.int32(1 << 30)),
                    axis=0, keepdims=True)
    cf_ref[0, 0] = jnp.where(conf > _CONF, conf, 0.0)
    cl_ref[0, 0] = cls_i.astype(jnp.float32)


def _nms_body(x1_ref, y1_ref, x2_ref, y2_ref, cf_ref, cl_ref, out_ref,
              vals_scr, sel_scr, *, rows, lw):
    vals_scr[...] = cf_ref[0]
    sel_scr[...] = jnp.zeros((8, _SELW), jnp.float32)

    idxarr = (jax.lax.broadcasted_iota(jnp.int32, (rows, lw), 0) * lw
              + jax.lax.broadcasted_iota(jnp.int32, (rows, lw), 1))
    lane = jax.lax.broadcasted_iota(jnp.int32, (1, lw), 1)
    lane_s = jax.lax.broadcasted_iota(jnp.int32, (1, _SELW), 1)

    def sel_body(i, _):
        v = vals_scr[...]
        m = jnp.max(v)
        n = jnp.min(jnp.where(v == m, idxarr, jnp.int32(1 << 30)))
        r = n // lw
        c = n % lw
        oh = lane == c
        row = vals_scr[pl.ds(r, 1), :]
        vals_scr[pl.ds(r, 1), :] = jnp.where(oh, -1.0, row)

        def ext(ref):
            return jnp.sum(jnp.where(oh, ref[0, pl.ds(r, 1), :], 0.0))

        ohs = lane_s == i
        for j, val in ((0, ext(x1_ref)), (1, ext(y1_ref)), (2, ext(x2_ref)),
                       (3, ext(y2_ref)), (4, m), (5, ext(cl_ref))):
            old = sel_scr[pl.ds(j, 1), :]
            sel_scr[pl.ds(j, 1), :] = jnp.where(ohs, val, old)
        return 0

    jax.lax.fori_loop(0, _MAXDET, sel_body, 0)

    sx1 = sel_scr[0:1, :]
    sy1 = sel_scr[1:2, :]
    sx2 = sel_scr[2:3, :]
    sy2 = sel_scr[3:4, :]
    sv = sel_scr[4:5, :]
    scl = sel_scr[5:6, :]

    off = scl * _MAXWH
    ox1 = sx1 + off
    oy1 = sy1 + off
    ox2 = sx2 + off
    oy2 = sy2 + off
    area = (sx2 - sx1) * (sy2 - sy1)

    keep0 = (sv > _CONF).astype(jnp.float32)

    def nms_step(i, keep):
        oh = lane_s == i

        def ext(v):
            return jnp.sum(jnp.where(oh, v, 0.0))

        x1i = ext(ox1)
        y1i = ext(oy1)
        x2i = ext(ox2)
        y2i = ext(oy2)
        ai = ext(area)
        ki = ext(keep)
        iw = jnp.maximum(jnp.minimum(x2i, ox2) - jnp.maximum(x1i, ox1), 0.0)
        ih = jnp.maximum(jnp.minimum(y2i, oy2) - jnp.maximum(y1i, oy1), 0.0)
        inter = iw * ih
        iou = inter / (ai + area - inter + 1e-9)
        suppress = ((iou > _IOU) & (lane_s > i) & (ki > 0.5)).astype(
            jnp.float32)
        return keep * (1.0 - suppress)

    kf = jax.lax.fori_loop(0, _MAXDET, nms_step, keep0)

    out_ref[0, 0:1, :] = sx1 * kf
    out_ref[0, 1:2, :] = sy1 * kf
    out_ref[0, 2:3, :] = sx2 * kf
    out_ref[0, 3:4, :] = sy2 * kf
    out_ref[0, 4:5, :] = sv * kf
    out_ref[0, 5:6, :] = scl * kf
    out_ref[0, 6:8, :] = jnp.zeros((2, _SELW), jnp.float32)


def kernel(preds, anchors, image_size):
    L, bs, C, H, W = preds.shape
    na = anchors.shape[1]
    nc = C // na
    HW = H * W
    NH = L * na
    N = NH * HW

    img = jnp.asarray(image_size, jnp.float32)
    sh = img / jnp.float32(H)
    sw = img / jnp.float32(W)
    aw = (anchors[..., 0] / sw).reshape(NH)
    ah = (anchors[..., 1] / sh).reshape(NH)
    params = jnp.stack(
        [aw, ah, jnp.broadcast_to(sw, (NH,)), jnp.broadcast_to(sh, (NH,))],
        axis=-1)

    p5 = preds.reshape(L, bs, na, nc, HW)

    decode = pl.pallas_call(
        functools.partial(_decode_body, nc=nc, H=H, W=W),
        grid=(L, bs, na),
        in_specs=[
            pl.BlockSpec(memory_space=pltpu.SMEM),
            pl.BlockSpec((1, 1, 1, nc, HW), lambda l, b, a: (l, b, a, 0, 0)),
        ],
        out_specs=[
            pl.BlockSpec((1, 1, 1, HW), lambda l, b, a: (b, l * na + a, 0, 0))
        ] * 6,
        out_shape=[jax.ShapeDtypeStruct((bs, NH, 1, HW), jnp.float32)] * 6,
    )
    x1, y1, x2, y2, cf, cl = decode(params, p5)

    lw = 128
    rows = N // lw
    def rsh(t):
        return t.reshape(bs, rows, lw)
    x1, y1, x2, y2, cf, cl = map(rsh, (x1, y1, x2, y2, cf, cl))

    nms = pl.pallas_call(
        functools.partial(_nms_body, rows=rows, lw=lw),
        grid=(bs,),
        in_specs=[pl.BlockSpec((1, rows, lw), lambda b: (b, 0, 0))] * 6,
        out_specs=pl.BlockSpec((1, 8, _SELW), lambda b: (b, 0, 0)),
        out_shape=jax.ShapeDtypeStruct((bs, 8, _SELW), jnp.float32),
        scratch_shapes=[
            pltpu.VMEM((rows, lw), jnp.float32),
            pltpu.VMEM((8, _SELW), jnp.float32),
        ],
    )
    det8 = nms(x1, y1, x2, y2, cf, cl)
    return det8[:, :6, :_MAXDET].transpose(0, 2, 1)


# speedup vs baseline: 3.7648x; 1.1235x over previous
import functools

import jax
import jax.numpy as jnp
from jax.experimental import pallas as pl
from jax.experimental.pallas import tpu as pltpu

_CONF = 0.2
_IOU = 0.6
_MAXDET = 300
_MAXWH = 4096.0
_SELW = 512


def _decode_body(params_ref, preds_ref, x1_ref, y1_ref, x2_ref, y2_ref,
                 cf_ref, cl_ref, *, nc, H, W):
    l = pl.program_id(0)
    a = pl.program_id(2)
    na = pl.num_programs(2)
    h = l * na + a
    aw = params_ref[h, 0]
    ah = params_ref[h, 1]
    sw = params_ref[h, 2]
    sh = params_ref[h, 3]

    p = preds_ref[0, 0, 0]
    s = jax.nn.sigmoid(p)

    ni = jax.lax.broadcasted_iota(jnp.int32, (1, H * W), 1)
    xg = (ni % W).astype(jnp.float32)
    yg = (ni // W).astype(jnp.float32)

    xc = (s[0:1, :] * 3.0 - 1.0 + xg) * sw
    yc = (s[1:2, :] * 3.0 - 1.0 + yg) * sh
    w = ((s[2:3, :] * 2.0) ** 2 * aw) * sw
    hh = ((s[3:4, :] * 2.0) ** 2 * ah) * sh

    x1_ref[0, 0] = xc - w / 2.0
    y1_ref[0, 0] = yc - hh / 2.0
    x2_ref[0, 0] = xc + w / 2.0
    y2_ref[0, 0] = yc + hh / 2.0

    obj = s[4:5, :]
    cls_s = s[5:, :] * obj
    conf = jnp.max(cls_s, axis=0, keepdims=True)
    kio = jax.lax.broadcasted_iota(jnp.int32, (nc - 5, H * W), 0)
    cls_i = jnp.min(jnp.where(cls_s == conf, kio, jnp.int32(1 << 30)),
                    axis=0, keepdims=True)
    cf_ref[0, 0] = jnp.where(conf > _CONF, conf, 0.0)
    cl_ref[0, 0] = cls_i.astype(jnp.float32)


_GRP = 48


def _nms_body(x1_ref, y1_ref, x2_ref, y2_ref, cf_ref, cl_ref, out_ref,
              vals_scr, gmax_scr, sn_scr, sb_scr, *, rows, lw):
    ngrp = rows // _GRP
    vals_scr[...] = cf_ref[0]
    for g in range(8):
        if g < ngrp:
            gmax_scr[g:g + 1, :] = jnp.max(
                cf_ref[0, g * _GRP:(g + 1) * _GRP, :], axis=0, keepdims=True)
        else:
            gmax_scr[g:g + 1, :] = jnp.full((1, lw), -1.0, jnp.float32)

    idxarr = (jax.lax.broadcasted_iota(jnp.int32, (rows, lw), 0) * lw
              + jax.lax.broadcasted_iota(jnp.int32, (rows, lw), 1))
    lane = jax.lax.broadcasted_iota(jnp.int32, (1, lw), 1)
    lane_s = jax.lax.broadcasted_iota(jnp.int32, (1, _SELW), 1)

    def sel_body(i, _):
        m = jnp.max(gmax_scr[...])
        v = vals_scr[...]
        n = jnp.min(jnp.where(v == m, idxarr, jnp.int32(1 << 30)))
        r = n // lw
        c = n % lw
        row = vals_scr[pl.ds(r, 1), :]
        vals_scr[pl.ds(r, 1), :] = jnp.where(lane == c, -1.0, row)
        g = r // _GRP
        blk = vals_scr[pl.ds(g * _GRP, _GRP), :]
        gmax_scr[pl.ds(g, 1), :] = jnp.max(blk, axis=0, keepdims=True)
        sn_scr[i] = n
        sb_scr[i, 4] = m
        return 0

    jax.lax.fori_loop(0, _MAXDET, sel_body, 0)

    def gat_body(t, vecs):
        for u in range(4):
            i = t * 4 + u
            n = sn_scr[i]
            r = n // lw
            c = n % lw
            oh = lane == c

            def ext(ref, r=r, oh=oh):
                return jnp.sum(jnp.where(oh, ref[0, pl.ds(r, 1), :], 0.0))

            scal = (ext(x1_ref), ext(y1_ref), ext(x2_ref), ext(y2_ref),
                    sb_scr[i, 4], ext(cl_ref))
            for j in (0, 1, 2, 3, 5):
                sb_scr[i, j] = scal[j]
            ohs = lane_s == i
            vecs = tuple(jnp.where(ohs, s, vec)
                         for s, vec in zip(scal, vecs))
        return vecs

    zero = jnp.zeros((1, _SELW), jnp.float32)
    sx1, sy1, sx2, sy2, sv, scl = jax.lax.fori_loop(
        0, _MAXDET // 4, gat_body, (zero,) * 6)

    off = scl * _MAXWH
    ox1 = sx1 + off
    oy1 = sy1 + off
    ox2 = sx2 + off
    oy2 = sy2 + off
    area = (ox2 - ox1) * (oy2 - oy1)

    keep0 = (sv > _CONF).astype(jnp.float32)

    def nms_step(i, keep):
        offi = sb_scr[i, 5] * _MAXWH
        ox1i = sb_scr[i, 0] + offi
        oy1i = sb_scr[i, 1] + offi
        ox2i = sb_scr[i, 2] + offi
        oy2i = sb_scr[i, 3] + offi
        ai = (ox2i - ox1i) * (oy2i - oy1i)
        ki = jnp.sum(jnp.where(lane_s == i, keep, 0.0))
        iw = jnp.maximum(jnp.minimum(ox2i, ox2) - jnp.maximum(ox1i, ox1), 0.0)
        ih = jnp.maximum(jnp.minimum(oy2i, oy2) - jnp.maximum(oy1i, oy1), 0.0)
        inter = iw * ih
        iou = inter / (ai + area - inter + 1e-9)
        suppress = ((iou > _IOU) & (lane_s > i) & (ki > 0.5)).astype(
            jnp.float32)
        return keep * (1.0 - suppress)

    kf = jax.lax.fori_loop(0, _MAXDET, nms_step, keep0)

    out_ref[0, 0:1, :] = sx1 * kf
    out_ref[0, 1:2, :] = sy1 * kf
    out_ref[0, 2:3, :] = sx2 * kf
    out_ref[0, 3:4, :] = sy2 * kf
    out_ref[0, 4:5, :] = sv * kf
    out_ref[0, 5:6, :] = scl * kf
    out_ref[0, 6:8, :] = jnp.zeros((2, _SELW), jnp.float32)


def kernel(preds, anchors, image_size):
    L, bs, C, H, W = preds.shape
    na = anchors.shape[1]
    nc = C // na
    HW = H * W
    NH = L * na
    N = NH * HW

    img = jnp.asarray(image_size, jnp.float32)
    sh = img / jnp.float32(H)
    sw = img / jnp.float32(W)
    aw = (anchors[..., 0] / sw).reshape(NH)
    ah = (anchors[..., 1] / sh).reshape(NH)
    params = jnp.stack(
        [aw, ah, jnp.broadcast_to(sw, (NH,)), jnp.broadcast_to(sh, (NH,))],
        axis=-1)

    p5 = preds.reshape(L, bs, na, nc, HW)

    decode = pl.pallas_call(
        functools.partial(_decode_body, nc=nc, H=H, W=W),
        grid=(L, bs, na),
        in_specs=[
            pl.BlockSpec(memory_space=pltpu.SMEM),
            pl.BlockSpec((1, 1, 1, nc, HW), lambda l, b, a: (l, b, a, 0, 0)),
        ],
        out_specs=[
            pl.BlockSpec((1, 1, 1, HW), lambda l, b, a: (b, l * na + a, 0, 0))
        ] * 6,
        out_shape=[jax.ShapeDtypeStruct((bs, NH, 1, HW), jnp.float32)] * 6,
    )
    x1, y1, x2, y2, cf, cl = decode(params, p5)

    lw = 128
    rows = N // lw
    def rsh(t):
        return t.reshape(bs, rows, lw)
    x1, y1, x2, y2, cf, cl = map(rsh, (x1, y1, x2, y2, cf, cl))

    nms = pl.pallas_call(
        functools.partial(_nms_body, rows=rows, lw=lw),
        grid=(bs,),
        in_specs=[pl.BlockSpec((1, rows, lw), lambda b: (b, 0, 0))] * 6,
        out_specs=pl.BlockSpec((1, 8, _SELW), lambda b: (b, 0, 0)),
        out_shape=jax.ShapeDtypeStruct((bs, 8, _SELW), jnp.float32),
        scratch_shapes=[
            pltpu.VMEM((rows, lw), jnp.float32),
            pltpu.VMEM((8, lw), jnp.float32),
            pltpu.SMEM((_MAXDET + 4,), jnp.int32),
            pltpu.SMEM((_MAXDET + 4, 8), jnp.float32),
        ],
    )
    det8 = nms(x1, y1, x2, y2, cf, cl)
    return det8[:, :6, :_MAXDET].transpose(0, 2, 1)
